# baseline (speedup 1.0000x reference)
import functools

import jax
import jax.numpy as jnp
from jax import lax
from jax.experimental import pallas as pl
from jax.experimental.pallas import tpu as pltpu
from jax.experimental.pallas import tpu_sc as plsc

N = 10000
E = 320000
NP = 10240
NC = 2
NS = 16
RPT = NP // NS
W = 200
BN = 1024
BND = 1024


def _make_sc_agg(D, core_feature_split, with_cnt, dtype=jnp.float32):
    mesh = plsc.VectorSubcoreMesh(core_axis_name="c", subcore_axis_name="s")
    if core_feature_split:
        out_type = [jax.ShapeDtypeStruct((NP, 2 * D), dtype)]
        ept = E // NS
    else:
        out_type = [jax.ShapeDtypeStruct((NC, NP, 128), dtype)]
        ept = E // (NC * NS)
    nw = ept // W
    assert nw % 2 == 0
    scratch = [
        pltpu.VMEM((W,), jnp.int32),
        pltpu.VMEM((W,), jnp.int32),
        pltpu.VMEM((W,), jnp.int32),
        pltpu.VMEM((W,), jnp.int32),
        pltpu.VMEM((W,), jnp.int32),
        pltpu.VMEM((W,), jnp.int32),
        pltpu.VMEM((W, D), dtype),
        pltpu.VMEM((W, D), dtype),
        pltpu.VMEM_SHARED((NP, D), dtype),
        pltpu.SemaphoreType.DMA,
        pltpu.SemaphoreType.DMA,
        pltpu.SemaphoreType.DMA,
        pltpu.SemaphoreType.DMA,
        pltpu.SemaphoreType.DMA,
        pltpu.SemaphoreType.DMA,
    ]
    if with_cnt:
        out_type.append(jax.ShapeDtypeStruct((NC, NP), jnp.float32))
        scratch += [
            pltpu.VMEM((W,), jnp.float32),
            pltpu.VMEM_SHARED((NP,), jnp.float32),
        ]

    def body(x_hbm, src_hbm, dst_hbm, z1_hbm, *rest):
        if with_cnt:
            (agg_hbm, cnt_hbm, srcA, dstA, srcB, dstB, gidxA, gidxB,
             rowsA, rowsB, agg_sh, semiA, semiB, semA, semB, semsA, semsB,
             ones_v, cnt_sh) = rest
        else:
            (agg_hbm, srcA, dstA, srcB, dstB, gidxA, gidxB,
             rowsA, rowsB, agg_sh, semiA, semiB, semA, semB,
             semsA, semsB) = rest
        cid = lax.axis_index("c")
        sid = lax.axis_index("s")
        if core_feature_split:
            base = sid * ept
        else:
            base = (cid * NS + sid) * ept

        def idx_issue(w, srcv, dstv, sem):
            pltpu.async_copy(src_hbm.at[pl.ds(base + w * W, W)], srcv, sem)
            pltpu.async_copy(dst_hbm.at[pl.ds(base + w * W, W)], dstv, sem)

        def idx_wait(srcv, dstv, sem):
            pltpu.make_async_copy(src_hbm.at[pl.ds(0, W)], srcv, sem).wait()
            pltpu.make_async_copy(dst_hbm.at[pl.ds(0, W)], dstv, sem).wait()

        def gidx_compute(srcv, gidxv):
            if not core_feature_split:
                return srcv
            for i in list(range(0, W - 15, 16)) + [W - 16]:
                s = pl.ds(i, 16)
                gidxv[s] = srcv[s] * 2 + cid
            return gidxv

        def gather_wait(rows, sem):
            pltpu.make_async_copy(x_hbm.at[pl.ds(0, W)], rows, sem).wait()

        def scatter_issue(rows, dstv, sems):
            pltpu.async_copy(rows, agg_sh.at[dstv], sems, add=True)
            if with_cnt:
                pltpu.async_copy(ones_v, cnt_sh.at[dstv], sems, add=True)

        def scatter_wait(rows, sems):
            pltpu.make_async_copy(x_hbm.at[pl.ds(0, W)], rows, sems).wait()
            if with_cnt:
                pltpu.make_async_copy(z1_hbm.at[pl.ds(0, W)], ones_v,
                                      sems).wait()

        if with_cnt:
            for i in list(range(0, W - 15, 16)) + [W - 16]:
                ones_v[pl.ds(i, 16)] = jnp.full((16,), 1.0, jnp.float32)

        zv = 32 if dtype == jnp.bfloat16 else 16
        zvec = jnp.zeros((zv,), dtype)
        for rr in range(160):
            for cc in range(0, D, zv):
                rowsA[rr, pl.ds(cc, zv)] = zvec
        for k in range(4):
            pltpu.sync_copy(rowsA.at[pl.ds(0, 160)],
                            agg_sh.at[pl.ds(sid * RPT + k * 160, 160)])
        if with_cnt:
            pltpu.sync_copy(z1_hbm.at[pl.ds(sid * RPT, RPT)],
                            cnt_sh.at[pl.ds(sid * RPT, RPT)])
        plsc.subcore_barrier()

        idx_issue(0, srcA, dstA, semiA)
        idx_wait(srcA, dstA, semiA)
        pltpu.async_copy(x_hbm.at[gidx_compute(srcA, gidxA)], rowsA, semA)
        idx_issue(1, srcB, dstB, semiB)

        @pl.loop(0, nw, step=2)
        def _(w):
            idx_wait(srcB, dstB, semiB)
            pltpu.async_copy(x_hbm.at[gidx_compute(srcB, gidxB)],
                             rowsB, semB)
            gather_wait(rowsA, semA)
            scatter_issue(rowsA, dstA, semsA)

            @pl.when(w + 2 < nw)
            def _():
                scatter_wait(rowsA, semsA)
                idx_issue(w + 2, srcA, dstA, semiA)

            @pl.when(w + 2 >= nw)
            def _():
                scatter_wait(rowsA, semsA)

            gather_wait(rowsB, semB)
            scatter_issue(rowsB, dstB, semsB)

            @pl.when(w + 2 < nw)
            def _():
                idx_wait(srcA, dstA, semiA)
                pltpu.async_copy(x_hbm.at[gidx_compute(srcA, gidxA)],
                                 rowsA, semA)

            @pl.when(w + 3 < nw)
            def _():
                scatter_wait(rowsB, semsB)
                idx_issue(w + 3, srcB, dstB, semiB)

            @pl.when(w + 3 >= nw)
            def _():
                scatter_wait(rowsB, semsB)

        plsc.subcore_barrier()

        rows_slice = pl.ds(sid * RPT, RPT)
        if core_feature_split:
            @pl.when(cid == 0)
            def _():
                pltpu.sync_copy(agg_sh.at[rows_slice],
                                agg_hbm.at[rows_slice, pl.ds(0, D)])

            @pl.when(cid == 1)
            def _():
                pltpu.sync_copy(agg_sh.at[rows_slice],
                                agg_hbm.at[rows_slice, pl.ds(D, D)])
        else:
            pltpu.sync_copy(agg_sh.at[rows_slice],
                            agg_hbm.at[cid, rows_slice, pl.ds(0, D)])
        if with_cnt:
            pltpu.sync_copy(cnt_sh.at[rows_slice],
                            cnt_hbm.at[cid, rows_slice])

    cp = pltpu.CompilerParams(use_tc_tiling_on_sc=False)
    return pl.kernel(body, mesh=mesh, out_type=out_type,
                     scratch_types=scratch, compiler_params=cp)


def _dotg(a, b):
    return lax.dot_general(a, b, (((1,), (1,)), ((), ())),
                           preferred_element_type=jnp.float32)


def _tc_skip_body(x_ref, b1l_ref, w1r_ref, xr_ref):
    xr_ref[...] = b1l_ref[...] + _dotg(x_ref[...], w1r_ref[...])


def _tc_layer_body(agg_ref, cnt_ref, xr_ref, w1l_ref,
                   w2lp_ref, w2rp_ref, b2lp_ref, p_ref, r_ref):
    a = (agg_ref[0].astype(jnp.float32)
         + agg_ref[1].astype(jnp.float32))
    c = cnt_ref[0] + cnt_ref[1]
    mean = a / jnp.clip(c, 1.0, None)[:, None]
    h = _dotg(mean, w1l_ref[...]) + xr_ref[...]
    h = jnp.maximum(h, 0.0)
    p_ref[...] = _dotg(h, w2lp_ref[...])
    r_ref[...] = _dotg(h, w2rp_ref[...]) + b2lp_ref[...]


def _tc_final_body(agg_ref, cnt_ref, r_ref, o_ref):
    a = agg_ref[0, :, :48] + agg_ref[1, :, :48]
    c = cnt_ref[0] + cnt_ref[1]
    res = (a / jnp.clip(c, 1.0, None)[:, None] + r_ref[...])[:, :40]
    o_ref[...] = res.T


def kernel(x, edge_index, W1l, b1l, W1r, W2l, b2l, W2r):
    x = x.astype(jnp.float32)
    ei = edge_index.astype(jnp.int32)
    src, dst = ei[0], ei[1]

    z1 = jnp.zeros((NP,), jnp.float32)

    w2lp = jnp.pad(W2l, ((0, 8), (0, 0)))
    w2rp = jnp.pad(W2r, ((0, 8), (0, 0)))
    b2lp = jnp.pad(b2l, (0, 8)).reshape(1, 48)
    b1l2 = b1l.reshape(1, 128)

    agg1, cnt = _make_sc_agg(128, False, True, jnp.bfloat16)(
        x.astype(jnp.bfloat16), src, dst, z1)

    xr = pl.pallas_call(
        _tc_skip_body,
        grid=(NP // BN,),
        in_specs=[
            pl.BlockSpec((BN, 128), lambda i: (i, 0)),
            pl.BlockSpec((1, 128), lambda i: (0, 0)),
            pl.BlockSpec((128, 128), lambda i: (0, 0)),
        ],
        out_specs=pl.BlockSpec((BN, 128), lambda i: (i, 0)),
        out_shape=jax.ShapeDtypeStruct((NP, 128), jnp.float32),
    )(x, b1l2, W1r)

    p, r = pl.pallas_call(
        _tc_layer_body,
        grid=(NP // BN,),
        in_specs=[
            pl.BlockSpec((NC, BN, 128), lambda i: (0, i, 0)),
            pl.BlockSpec((NC, BN), lambda i: (0, i)),
            pl.BlockSpec((BN, 128), lambda i: (i, 0)),
            pl.BlockSpec((128, 128), lambda i: (0, 0)),
            pl.BlockSpec((48, 128), lambda i: (0, 0)),
            pl.BlockSpec((48, 128), lambda i: (0, 0)),
            pl.BlockSpec((1, 48), lambda i: (0, 0)),
        ],
        out_specs=[
            pl.BlockSpec((BN, 48), lambda i: (i, 0)),
            pl.BlockSpec((BN, 48), lambda i: (i, 0)),
        ],
        out_shape=[
            jax.ShapeDtypeStruct((NP, 48), jnp.float32),
            jax.ShapeDtypeStruct((NP, 48), jnp.float32),
        ],
    )(agg1, cnt, xr, W1l, w2lp, w2rp, b2lp)

    (agg2,) = _make_sc_agg(48, False, False)(p, src, dst, z1)

    out = pl.pallas_call(
        _tc_final_body,
        grid=(NP // BND,),
        in_specs=[
            pl.BlockSpec((NC, BND, 128), lambda i: (0, i, 0)),
            pl.BlockSpec((NC, BND), lambda i: (0, i)),
            pl.BlockSpec((BND, 48), lambda i: (i, 0)),
        ],
        out_specs=pl.BlockSpec((40, BND), lambda i: (0, i)),
        out_shape=jax.ShapeDtypeStruct((40, N), jnp.float32),
    )(agg2, cnt, r)

    return out.T


# speedup vs baseline: 1.0017x; 1.0017x over previous
import functools

import jax
import jax.numpy as jnp
from jax import lax
from jax.experimental import pallas as pl
from jax.experimental.pallas import tpu as pltpu
from jax.experimental.pallas import tpu_sc as plsc

N = 10000
E = 320000
NP = 10240
NC = 2
NS = 16
RPT = NP // NS
W = 200
BN = 1024
BND = 1024


def _make_sc_agg(D, core_feature_split, with_cnt, dtype=jnp.float32):
    mesh = plsc.VectorSubcoreMesh(core_axis_name="c", subcore_axis_name="s")
    if core_feature_split:
        out_type = [jax.ShapeDtypeStruct((NP, 2 * D), dtype)]
        ept = E // NS
    else:
        out_type = [jax.ShapeDtypeStruct((NC, NP, 128), dtype)]
        ept = E // (NC * NS)
    nw = ept // W
    assert nw % 2 == 0
    scratch = [
        pltpu.VMEM((W,), jnp.int32),
        pltpu.VMEM((W,), jnp.int32),
        pltpu.VMEM((W,), jnp.int32),
        pltpu.VMEM((W,), jnp.int32),
        pltpu.VMEM((W,), jnp.int32),
        pltpu.VMEM((W,), jnp.int32),
        pltpu.VMEM((W, D), dtype),
        pltpu.VMEM((W, D), dtype),
        pltpu.VMEM_SHARED((NP, D), dtype),
        pltpu.SemaphoreType.DMA,
        pltpu.SemaphoreType.DMA,
        pltpu.SemaphoreType.DMA,
        pltpu.SemaphoreType.DMA,
        pltpu.SemaphoreType.DMA,
        pltpu.SemaphoreType.DMA,
    ]
    if with_cnt:
        out_type.append(jax.ShapeDtypeStruct((NC, NP), jnp.float32))
        scratch += [
            pltpu.VMEM((W,), jnp.float32),
            pltpu.VMEM_SHARED((NP,), jnp.float32),
        ]

    def body(x_hbm, src_hbm, dst_hbm, z1_hbm, *rest):
        if with_cnt:
            (agg_hbm, cnt_hbm, srcA, dstA, srcB, dstB, gidxA, gidxB,
             rowsA, rowsB, agg_sh, semiA, semiB, semA, semB, semsA, semsB,
             ones_v, cnt_sh) = rest
        else:
            (agg_hbm, srcA, dstA, srcB, dstB, gidxA, gidxB,
             rowsA, rowsB, agg_sh, semiA, semiB, semA, semB,
             semsA, semsB) = rest
        cid = lax.axis_index("c")
        sid = lax.axis_index("s")
        if core_feature_split:
            base = sid * ept
        else:
            base = (cid * NS + sid) * ept

        def idx_issue(w, srcv, dstv, sem):
            pltpu.async_copy(src_hbm.at[pl.ds(base + w * W, W)], srcv, sem)
            pltpu.async_copy(dst_hbm.at[pl.ds(base + w * W, W)], dstv, sem)

        def idx_wait(srcv, dstv, sem):
            pltpu.make_async_copy(src_hbm.at[pl.ds(0, W)], srcv, sem).wait()
            pltpu.make_async_copy(dst_hbm.at[pl.ds(0, W)], dstv, sem).wait()

        def gidx_compute(srcv, gidxv):
            if not core_feature_split:
                return srcv
            for i in list(range(0, W - 15, 16)) + [W - 16]:
                s = pl.ds(i, 16)
                gidxv[s] = srcv[s] * 2 + cid
            return gidxv

        def gather_wait(rows, sem):
            pltpu.make_async_copy(x_hbm.at[pl.ds(0, W)], rows, sem).wait()

        def scatter_issue(rows, dstv, sems):
            pltpu.async_copy(rows, agg_sh.at[dstv], sems, add=True)
            if with_cnt:
                pltpu.async_copy(ones_v, cnt_sh.at[dstv], sems, add=True)

        def scatter_wait(rows, sems):
            pltpu.make_async_copy(x_hbm.at[pl.ds(0, W)], rows, sems).wait()
            if with_cnt:
                pltpu.make_async_copy(z1_hbm.at[pl.ds(0, W)], ones_v,
                                      sems).wait()

        if with_cnt:
            for i in list(range(0, W - 15, 16)) + [W - 16]:
                ones_v[pl.ds(i, 16)] = jnp.full((16,), 1.0, jnp.float32)

        zv = 32 if dtype == jnp.bfloat16 else 16
        zvec = jnp.zeros((zv,), dtype)
        for rr in range(160):
            for cc in range(0, D, zv):
                rowsA[rr, pl.ds(cc, zv)] = zvec
        for k in range(4):
            pltpu.sync_copy(rowsA.at[pl.ds(0, 160)],
                            agg_sh.at[pl.ds(sid * RPT + k * 160, 160)])
        if with_cnt:
            pltpu.sync_copy(z1_hbm.at[pl.ds(sid * RPT, RPT)],
                            cnt_sh.at[pl.ds(sid * RPT, RPT)])
        plsc.subcore_barrier()

        idx_issue(0, srcA, dstA, semiA)
        idx_wait(srcA, dstA, semiA)
        pltpu.async_copy(x_hbm.at[gidx_compute(srcA, gidxA)], rowsA, semA)
        idx_issue(1, srcB, dstB, semiB)

        @pl.loop(0, nw, step=2)
        def _(w):
            idx_wait(srcB, dstB, semiB)
            pltpu.async_copy(x_hbm.at[gidx_compute(srcB, gidxB)],
                             rowsB, semB)
            gather_wait(rowsA, semA)
            scatter_issue(rowsA, dstA, semsA)

            @pl.when(w + 2 < nw)
            def _():
                scatter_wait(rowsA, semsA)
                idx_issue(w + 2, srcA, dstA, semiA)

            @pl.when(w + 2 >= nw)
            def _():
                scatter_wait(rowsA, semsA)

            gather_wait(rowsB, semB)
            scatter_issue(rowsB, dstB, semsB)

            @pl.when(w + 2 < nw)
            def _():
                idx_wait(srcA, dstA, semiA)
                pltpu.async_copy(x_hbm.at[gidx_compute(srcA, gidxA)],
                                 rowsA, semA)

            @pl.when(w + 3 < nw)
            def _():
                scatter_wait(rowsB, semsB)
                idx_issue(w + 3, srcB, dstB, semiB)

            @pl.when(w + 3 >= nw)
            def _():
                scatter_wait(rowsB, semsB)

        plsc.subcore_barrier()

        rows_slice = pl.ds(sid * RPT, RPT)
        if core_feature_split:
            @pl.when(cid == 0)
            def _():
                pltpu.sync_copy(agg_sh.at[rows_slice],
                                agg_hbm.at[rows_slice, pl.ds(0, D)])

            @pl.when(cid == 1)
            def _():
                pltpu.sync_copy(agg_sh.at[rows_slice],
                                agg_hbm.at[rows_slice, pl.ds(D, D)])
        else:
            pltpu.sync_copy(agg_sh.at[rows_slice],
                            agg_hbm.at[cid, rows_slice, pl.ds(0, D)])
        if with_cnt:
            pltpu.sync_copy(cnt_sh.at[rows_slice],
                            cnt_hbm.at[cid, rows_slice])

    cp = pltpu.CompilerParams(use_tc_tiling_on_sc=False)
    return pl.kernel(body, mesh=mesh, out_type=out_type,
                     scratch_types=scratch, compiler_params=cp)


def _dotg(a, b):
    return lax.dot_general(a, b, (((1,), (1,)), ((), ())),
                           preferred_element_type=jnp.float32)


def _tc_layer_body(agg_ref, cnt_ref, x_ref, w1l_ref, b1l_ref, w1r_ref,
                   w2lp_ref, w2rp_ref, b2lp_ref, p_ref, r_ref):
    a = (agg_ref[0].astype(jnp.float32)
         + agg_ref[1].astype(jnp.float32))
    c = cnt_ref[0] + cnt_ref[1]
    mean = a / jnp.clip(c, 1.0, None)[:, None]
    h = (_dotg(mean, w1l_ref[...]) + b1l_ref[...]
         + _dotg(x_ref[...], w1r_ref[...]))
    h = jnp.maximum(h, 0.0)
    p_ref[...] = _dotg(h, w2lp_ref[...])
    r_ref[...] = _dotg(h, w2rp_ref[...]) + b2lp_ref[...]


def _tc_final_body(agg_ref, cnt_ref, r_ref, o_ref):
    a = agg_ref[0, :, :48] + agg_ref[1, :, :48]
    c = cnt_ref[0] + cnt_ref[1]
    res = (a / jnp.clip(c, 1.0, None)[:, None] + r_ref[...])[:, :40]
    o_ref[...] = res.T


def kernel(x, edge_index, W1l, b1l, W1r, W2l, b2l, W2r):
    x = x.astype(jnp.float32)
    ei = edge_index.astype(jnp.int32)
    src, dst = ei[0], ei[1]

    z1 = jnp.zeros((NP,), jnp.float32)

    w2lp = jnp.pad(W2l, ((0, 8), (0, 0)))
    w2rp = jnp.pad(W2r, ((0, 8), (0, 0)))
    b2lp = jnp.pad(b2l, (0, 8)).reshape(1, 48)
    b1l2 = b1l.reshape(1, 128)

    agg1, cnt = _make_sc_agg(128, False, True, jnp.bfloat16)(
        x.astype(jnp.bfloat16), src, dst, z1)

    p, r = pl.pallas_call(
        _tc_layer_body,
        grid=(NP // BN,),
        in_specs=[
            pl.BlockSpec((NC, BN, 128), lambda i: (0, i, 0)),
            pl.BlockSpec((NC, BN), lambda i: (0, i)),
            pl.BlockSpec((BN, 128), lambda i: (i, 0)),
            pl.BlockSpec((128, 128), lambda i: (0, 0)),
            pl.BlockSpec((1, 128), lambda i: (0, 0)),
            pl.BlockSpec((128, 128), lambda i: (0, 0)),
            pl.BlockSpec((48, 128), lambda i: (0, 0)),
            pl.BlockSpec((48, 128), lambda i: (0, 0)),
            pl.BlockSpec((1, 48), lambda i: (0, 0)),
        ],
        out_specs=[
            pl.BlockSpec((BN, 48), lambda i: (i, 0)),
            pl.BlockSpec((BN, 48), lambda i: (i, 0)),
        ],
        out_shape=[
            jax.ShapeDtypeStruct((NP, 48), jnp.float32),
            jax.ShapeDtypeStruct((NP, 48), jnp.float32),
        ],
    )(agg1, cnt, x, W1l, b1l2, W1r, w2lp, w2rp, b2lp)

    (agg2,) = _make_sc_agg(48, False, False)(p, src, dst, z1)

    out = pl.pallas_call(
        _tc_final_body,
        grid=(NP // BND,),
        in_specs=[
            pl.BlockSpec((NC, BND, 128), lambda i: (0, i, 0)),
            pl.BlockSpec((NC, BND), lambda i: (0, i)),
            pl.BlockSpec((BND, 48), lambda i: (i, 0)),
        ],
        out_specs=pl.BlockSpec((40, BND), lambda i: (0, i)),
        out_shape=jax.ShapeDtypeStruct((40, N), jnp.float32),
    )(agg2, cnt, r)

    return out.T
